# baseline (speedup 1.0000x reference)
import functools

import jax
import jax.numpy as jnp
from jax import lax
from jax.experimental import pallas as pl
from jax.experimental.pallas import tpu as pltpu
from jax.experimental.pallas import tpu_sc as plsc

_NUM_WORKERS = 32
_IDX_ROW = 128
_CHUNK = 1024


def _make_gather(num_idx_rows: int, d: int):
  mesh = plsc.VectorSubcoreMesh(core_axis_name="c", subcore_axis_name="s")
  chunk_rows = _CHUNK // _IDX_ROW

  @functools.partial(
      pl.kernel,
      mesh=mesh,
      out_type=jax.ShapeDtypeStruct((num_idx_rows * _IDX_ROW, d),
                                    jnp.float32),
      scratch_types=[
          pltpu.VMEM((chunk_rows, _IDX_ROW), jnp.int32),
          pltpu.VMEM((_CHUNK, d), jnp.float32),
          pltpu.SemaphoreType.DMA,
      ],
      compiler_params=pltpu.CompilerParams(use_tc_tiling_on_sc=False),
  )
  def gather_kernel(ids_hbm, table_hbm, out_hbm, idx_v, rows_v, sem):
    cid = lax.axis_index("c")
    sid = lax.axis_index("s")
    wid = sid * 2 + cid
    rows_per_w = num_idx_rows // _NUM_WORKERS
    n_chunks = rows_per_w // chunk_rows
    base_idx_row = wid * rows_per_w

    def body(g, carry):
      r0 = base_idx_row + g * chunk_rows
      pltpu.sync_copy(ids_hbm.at[pl.ds(r0, chunk_rows)], idx_v)
      copies = []
      for j in range(chunk_rows):
        copies.append(
            pltpu.async_copy(
                table_hbm.at[idx_v.at[j]],
                rows_v.at[pl.ds(j * _IDX_ROW, _IDX_ROW)],
                sem,
            ))
      for c in copies:
        c.wait()
      pltpu.sync_copy(rows_v, out_hbm.at[pl.ds(r0 * _IDX_ROW, _CHUNK)])
      return carry

    lax.fori_loop(0, n_chunks, body, 0)

  return gather_kernel


def kernel(token_ids, E):
  bsz, seq = token_ids.shape
  _, d = E.shape
  n = bsz * seq
  ids = token_ids.reshape(n // _IDX_ROW, _IDX_ROW).astype(jnp.int32)
  out = _make_gather(n // _IDX_ROW, d)(ids, E)
  return out.reshape(bsz, seq, d)


# speedup vs baseline: 1.0099x; 1.0099x over previous
import functools

import jax
import jax.numpy as jnp
from jax import lax
from jax.experimental import pallas as pl
from jax.experimental.pallas import tpu as pltpu
from jax.experimental.pallas import tpu_sc as plsc

_NUM_WORKERS = 32
_IDX_ROW = 128
_CHUNK = 512
_SPD = _CHUNK // _IDX_ROW


def _make_gather(num_idx_rows: int, d: int):
  mesh = plsc.VectorSubcoreMesh(core_axis_name="c", subcore_axis_name="s")
  rows_per_w = num_idx_rows // _NUM_WORKERS

  @functools.partial(
      pl.kernel,
      mesh=mesh,
      out_type=jax.ShapeDtypeStruct((num_idx_rows * _IDX_ROW, d),
                                    jnp.float32),
      scratch_types=[
          pltpu.VMEM((rows_per_w, _IDX_ROW), jnp.int32),
          pltpu.VMEM((_CHUNK, d), jnp.float32),
          pltpu.VMEM((_CHUNK, d), jnp.float32),
          pltpu.SemaphoreType.DMA,
          pltpu.SemaphoreType.DMA,
          pltpu.SemaphoreType.DMA,
          pltpu.SemaphoreType.DMA,
      ],
      compiler_params=pltpu.CompilerParams(use_tc_tiling_on_sc=False),
  )
  def gather_kernel(ids_hbm, table_hbm, out_hbm, idx_v, buf0, buf1,
                    g0, g1, s0, s1):
    cid = lax.axis_index("c")
    sid = lax.axis_index("s")
    wid = sid * 2 + cid
    base_idx_row = wid * rows_per_w
    n_chunks = rows_per_w // _SPD

    pltpu.sync_copy(ids_hbm.at[pl.ds(base_idx_row, rows_per_w)], idx_v)

    def fire(g, buf, sem):
      for j in range(_SPD):
        pltpu.async_copy(
            table_hbm.at[idx_v.at[g * _SPD + j]],
            buf.at[pl.ds(j * _IDX_ROW, _IDX_ROW)],
            sem,
        )

    def wait_fire(buf, sem):
      for _ in range(_SPD):
        pltpu.make_async_copy(
            table_hbm.at[idx_v.at[0]],
            buf.at[pl.ds(0, _IDX_ROW)],
            sem,
        ).wait()

    def store(g, buf, sem):
      row0 = (base_idx_row + g * _SPD) * _IDX_ROW
      return pltpu.async_copy(buf, out_hbm.at[pl.ds(row0, _CHUNK)], sem)

    def wait_store(g, buf, sem):
      row0 = (base_idx_row + g * _SPD) * _IDX_ROW
      pltpu.make_async_copy(buf, out_hbm.at[pl.ds(row0, _CHUNK)], sem).wait()

    fire(0, buf0, g0)
    fire(1, buf1, g1)

    def body(i, carry):
      c0 = 2 * i
      wait_fire(buf0, g0)
      store(c0, buf0, s0)
      wait_fire(buf1, g1)
      store(c0 + 1, buf1, s1)
      wait_store(c0, buf0, s0)

      @pl.when(i < n_chunks // 2 - 1)
      def _():
        fire(c0 + 2, buf0, g0)

      wait_store(c0 + 1, buf1, s1)

      @pl.when(i < n_chunks // 2 - 1)
      def _():
        fire(c0 + 3, buf1, g1)

      return carry

    lax.fori_loop(0, n_chunks // 2, body, 0)

  return gather_kernel


def kernel(token_ids, E):
  bsz, seq = token_ids.shape
  _, d = E.shape
  n = bsz * seq
  ids = token_ids.reshape(n // _IDX_ROW, _IDX_ROW).astype(jnp.int32)
  out = _make_gather(n // _IDX_ROW, d)(ids, E)
  return out.reshape(bsz, seq, d)


# speedup vs baseline: 1.0102x; 1.0002x over previous
import functools

import jax
import jax.numpy as jnp
from jax import lax
from jax.experimental import pallas as pl
from jax.experimental.pallas import tpu as pltpu
from jax.experimental.pallas import tpu_sc as plsc

_NUM_WORKERS = 32
_IDX_ROW = 128
_CHUNK = 512
_SPD = _CHUNK // _IDX_ROW


def _make_gather(num_idx_rows: int, d: int):
  mesh = plsc.VectorSubcoreMesh(core_axis_name="c", subcore_axis_name="s")
  rows_per_w = num_idx_rows // _NUM_WORKERS

  @functools.partial(
      pl.kernel,
      mesh=mesh,
      out_type=jax.ShapeDtypeStruct((num_idx_rows * _IDX_ROW, d),
                                    jnp.float32),
      scratch_types=[
          pltpu.VMEM((rows_per_w, _IDX_ROW), jnp.int32),
          pltpu.VMEM((_CHUNK, d), jnp.float32),
          pltpu.VMEM((_CHUNK, d), jnp.float32),
          pltpu.SemaphoreType.DMA,
          pltpu.SemaphoreType.DMA,
          pltpu.SemaphoreType.DMA,
          pltpu.SemaphoreType.DMA,
      ],
      compiler_params=pltpu.CompilerParams(use_tc_tiling_on_sc=False),
      cost_estimate=pl.CostEstimate(
          flops=0,
          transcendentals=0,
          bytes_accessed=num_idx_rows * _IDX_ROW * (4 + 8 * d),
      ),
  )
  def gather_kernel(ids_hbm, table_hbm, out_hbm, idx_v, buf0, buf1,
                    g0, g1, s0, s1):
    cid = lax.axis_index("c")
    sid = lax.axis_index("s")
    wid = sid * 2 + cid
    base_idx_row = wid * rows_per_w
    n_chunks = rows_per_w // _SPD

    pltpu.sync_copy(ids_hbm.at[pl.ds(base_idx_row, rows_per_w)], idx_v)

    def fire(g, buf, sem):
      for j in range(_SPD):
        pltpu.async_copy(
            table_hbm.at[idx_v.at[g * _SPD + j]],
            buf.at[pl.ds(j * _IDX_ROW, _IDX_ROW)],
            sem,
        )

    def wait_fire(buf, sem):
      for _ in range(_SPD):
        pltpu.make_async_copy(
            table_hbm.at[idx_v.at[0]],
            buf.at[pl.ds(0, _IDX_ROW)],
            sem,
        ).wait()

    def store(g, buf, sem):
      row0 = (base_idx_row + g * _SPD) * _IDX_ROW
      return pltpu.async_copy(buf, out_hbm.at[pl.ds(row0, _CHUNK)], sem)

    def wait_store(g, buf, sem):
      row0 = (base_idx_row + g * _SPD) * _IDX_ROW
      pltpu.make_async_copy(buf, out_hbm.at[pl.ds(row0, _CHUNK)], sem).wait()

    fire(0, buf0, g0)
    fire(1, buf1, g1)

    def body(i, carry):
      c0 = 2 * i
      wait_fire(buf0, g0)
      store(c0, buf0, s0)
      wait_fire(buf1, g1)
      store(c0 + 1, buf1, s1)
      wait_store(c0, buf0, s0)

      @pl.when(i < n_chunks // 2 - 1)
      def _():
        fire(c0 + 2, buf0, g0)

      wait_store(c0 + 1, buf1, s1)

      @pl.when(i < n_chunks // 2 - 1)
      def _():
        fire(c0 + 3, buf1, g1)

      return carry

    lax.fori_loop(0, n_chunks // 2, body, 0)

  return gather_kernel


def kernel(token_ids, E):
  bsz, seq = token_ids.shape
  _, d = E.shape
  n = bsz * seq
  ids = token_ids.reshape(n // _IDX_ROW, _IDX_ROW).astype(jnp.int32)
  out = _make_gather(n // _IDX_ROW, d)(ids, E)
  return out.reshape(bsz, seq, d)
